# baseline (speedup 1.0000x reference)
import functools

import jax
import jax.numpy as jnp
from jax import lax
from jax.experimental import pallas as pl
from jax.experimental.pallas import tpu as pltpu
from jax.experimental.pallas import tpu_sc as plsc

N = 10000
E = 320000
D = 128
NP = 10240

NC = 2
NS = 16
NW = NC * NS
EPW = E // NW
K = 50
ITERS = EPW // K
RPW = NP // NS
HR = NP // D

_mesh = plsc.VectorSubcoreMesh(
    core_axis_name="c", subcore_axis_name="s", num_cores=NC, num_subcores=NS
)


@functools.partial(
    pl.kernel,
    out_type=jax.ShapeDtypeStruct((NC, HR, D), jnp.float32),
    mesh=_mesh,
    compiler_params=pltpu.CompilerParams(needs_layout_passes=False),
    scratch_types=[
        pltpu.VMEM((EPW,), jnp.int32),
        pltpu.VMEM((HR, D), jnp.float32),
        pltpu.VMEM((1, HR), jnp.int32),
        pltpu.VMEM_SHARED((HR, D), jnp.float32),
    ],
)
def _sc_degree(dst_hbm, iota_hbm, zeros_hbm, out_hbm, idx_v, hist_v, iota_v, deg_sh):
    c = lax.axis_index("c")
    s = lax.axis_index("s")
    wid = s * NC + c
    e0 = pl.multiple_of(wid * EPW, 8)
    pltpu.sync_copy(dst_hbm.at[pl.ds(e0, EPW)], idx_v)
    pltpu.sync_copy(zeros_hbm.at[pl.ds(0, HR)], hist_v)
    pltpu.sync_copy(iota_hbm, iota_v)

    @pl.when(s == 0)
    def _():
        pltpu.sync_copy(zeros_hbm.at[pl.ds(0, HR)], deg_sh)

    def body(v, carry):
        idx16 = idx_v[pl.ds(v * 16, 16)]
        row = jnp.right_shift(idx16, 7)
        col = jnp.bitwise_and(idx16, 127)
        cnt, last = plsc.scan_count(idx16)
        plsc.addupdate_scatter(hist_v, [row, col], cnt.astype(jnp.float32), mask=last)
        return carry

    lax.fori_loop(0, EPW // 16, body, 0)
    plsc.subcore_barrier()
    pltpu.sync_copy(hist_v, deg_sh.at[iota_v.at[0]], add=True)
    plsc.subcore_barrier()

    @pl.when(s == 0)
    def _():
        pltpu.sync_copy(deg_sh, out_hbm.at[c])


BC = 8
NB = ITERS // BC
IB = 2
NBUF = 4


@functools.partial(
    pl.kernel,
    out_type=jax.ShapeDtypeStruct((NC, NP, D), jnp.float32),
    mesh=_mesh,
    scratch_types=[
        pltpu.VMEM((IB, BC, K), jnp.int32),
        pltpu.VMEM((IB, BC, K), jnp.int32),
        [pltpu.VMEM((K, D), jnp.float32)] * NBUF,
        [pltpu.SemaphoreType.DMA] * NBUF,
        [pltpu.SemaphoreType.DMA] * NBUF,
        pltpu.VMEM_SHARED((NP, D), jnp.float32),
        pltpu.SemaphoreType.DMA,
    ],
)
def _sc_aggregate(
    z_hbm, src2d_hbm, dst2d_hbm, zerosd_hbm, out_hbm,
    sring, dring, rows, gsems, ssems, acc_sh, isem
):
    c = lax.axis_index("c")
    s = lax.axis_index("s")
    wid = s * NC + c
    chunk0 = pl.multiple_of(wid * ITERS, 8)
    row0 = pl.multiple_of(s * RPW, 8)

    def fire_idx(b, sl):
        blk = pl.multiple_of(chunk0 + b * BC, 8)
        pltpu.async_copy(src2d_hbm.at[pl.ds(blk, BC)], sring.at[sl], isem)
        pltpu.async_copy(dst2d_hbm.at[pl.ds(blk, BC)], dring.at[sl], isem)

    def drain_idx():
        pltpu.make_async_copy(src2d_hbm.at[pl.ds(chunk0, BC)], sring.at[0], isem).wait()
        pltpu.make_async_copy(dst2d_hbm.at[pl.ds(chunk0, BC)], dring.at[0], isem).wait()

    def wait_scatter(q):
        pltpu.make_async_copy(rows[q], acc_sh.at[dring.at[0, 0]], ssems[q]).wait()

    def wait_gather(q):
        pltpu.make_async_copy(z_hbm.at[sring.at[0, 0]], rows[q], gsems[q]).wait()

    pltpu.sync_copy(zerosd_hbm, acc_sh.at[pl.ds(row0, RPW)])
    fire_idx(0, 0)
    fire_idx(1, 1)
    plsc.subcore_barrier()
    drain_idx()
    pltpu.async_copy(z_hbm.at[sring.at[0, 0]], rows[0], gsems[0])

    def block_body(b, carry):
        sl = lax.rem(b, IB)
        nsl = lax.rem(b + 1, IB)

        @pl.when(b + 1 < NB)
        def _():
            drain_idx()

        for r in range(BC):
            q = r % NBUF
            nq = (r + 1) % NBUF

            if r < NBUF - 1:
                @pl.when(b > 0)
                def _():
                    wait_scatter(nq)
            else:
                wait_scatter(nq)
            if r < BC - 1:
                pltpu.async_copy(z_hbm.at[sring.at[sl, r + 1]], rows[nq], gsems[nq])
            else:
                pltpu.async_copy(z_hbm.at[sring.at[nsl, 0]], rows[nq], gsems[nq])
            wait_gather(q)
            pltpu.async_copy(rows[q], acc_sh.at[dring.at[sl, r]], ssems[q], add=True)

        @pl.when(b + 2 < NB)
        def _():
            fire_idx(b + 2, sl)

        return carry

    lax.fori_loop(0, NB, block_body, 0)
    for j in range(ITERS - NBUF + 1, ITERS):
        wait_scatter(j % NBUF)
    wait_gather(ITERS % NBUF)
    plsc.subcore_barrier()
    pltpu.sync_copy(acc_sh.at[pl.ds(row0, RPW)], out_hbm.at[c, pl.ds(row0, RPW)])


BR = 2048

_DOT = dict(precision=lax.Precision.HIGHEST, preferred_element_type=jnp.float32)


def _gelu(x):
    return 0.5 * x * (1.0 + lax.erf(x * 0.7071067811865476))


def _tc_proj_body(degcol, latent, wp, bp, w1, z1_out, dinv_out):
    deg = jnp.sum(degcol[...], axis=0) + 1.0
    dinv = lax.rsqrt(deg)
    x0 = _gelu(jnp.dot(latent[...], wp[...], **_DOT) + bp[...])
    z1_out[...] = jnp.dot(x0, w1[...], **_DOT) * dinv
    dinv_out[...] = jnp.broadcast_to(dinv, (BR, D))


def _tc_proj(degcol, latent, wp, bp, w1):
    return pl.pallas_call(
        _tc_proj_body,
        grid=(NP // BR,),
        in_specs=[
            pl.BlockSpec((NC, BR, 1), lambda i: (0, i, 0)),
            pl.BlockSpec((BR, D), lambda i: (i, 0)),
            pl.BlockSpec((D, D), lambda i: (0, 0)),
            pl.BlockSpec((1, D), lambda i: (0, 0)),
            pl.BlockSpec((D, D), lambda i: (0, 0)),
        ],
        out_specs=[
            pl.BlockSpec((BR, D), lambda i: (i, 0)),
            pl.BlockSpec((BR, D), lambda i: (i, 0)),
        ],
        out_shape=[
            jax.ShapeDtypeStruct((NP, D), jnp.float32),
            jax.ShapeDtypeStruct((NP, D), jnp.float32),
        ],
    )(degcol, latent, wp, bp, w1)


def _tc_combine_body(acc, z, dinv, b, w_next, znext_out):
    x = dinv[...] * (acc[0] + acc[1] + z[...]) + b[...]
    znext_out[...] = jnp.dot(x, w_next[...], **_DOT) * dinv[...]


def _tc_combine(acc, z, dinv, b, w_next):
    return pl.pallas_call(
        _tc_combine_body,
        grid=(NP // BR,),
        in_specs=[
            pl.BlockSpec((NC, BR, D), lambda i: (0, i, 0)),
            pl.BlockSpec((BR, D), lambda i: (i, 0)),
            pl.BlockSpec((BR, D), lambda i: (i, 0)),
            pl.BlockSpec((1, D), lambda i: (0, 0)),
            pl.BlockSpec((D, D), lambda i: (0, 0)),
        ],
        out_specs=pl.BlockSpec((BR, D), lambda i: (i, 0)),
        out_shape=jax.ShapeDtypeStruct((NP, D), jnp.float32),
    )(acc, z, dinv, b, w_next)


def _tc_final_body(acc, z, dinv, b, out):
    out[...] = dinv[...] * (acc[0] + acc[1] + z[...]) + b[...]


def _tc_final(acc, z, dinv, b):
    return pl.pallas_call(
        _tc_final_body,
        grid=(NP // BR,),
        in_specs=[
            pl.BlockSpec((NC, BR, D), lambda i: (0, i, 0)),
            pl.BlockSpec((BR, D), lambda i: (i, 0)),
            pl.BlockSpec((BR, D), lambda i: (i, 0)),
            pl.BlockSpec((1, D), lambda i: (0, 0)),
        ],
        out_specs=pl.BlockSpec((BR, D), lambda i: (i, 0)),
        out_shape=jax.ShapeDtypeStruct((NP, D), jnp.float32),
    )(acc, z, dinv, b)


def kernel(latent, edge_index, Wp, bp, W1, b1, W2, b2, W3, b3):
    src2d = edge_index[0].reshape(E // K, K)
    dst2d = edge_index[1].reshape(E // K, K)
    dstflat = edge_index[1]
    iota_hr = jnp.arange(HR, dtype=jnp.int32).reshape(1, HR)
    zerosd = jnp.zeros((RPW, D), jnp.float32)
    latp = jnp.pad(latent, ((0, NP - N), (0, 0)))
    bp2, b12, b22, b32 = (b.reshape(1, D) for b in (bp, b1, b2, b3))

    degpart = _sc_degree(dstflat, iota_hr, zerosd)
    degcol = degpart.reshape(NC, NP, 1)
    z1, dinv = _tc_proj(degcol, latp, Wp, bp2, W1)
    acc1 = _sc_aggregate(z1, src2d, dst2d, zerosd)
    z2 = _tc_combine(acc1, z1, dinv, b12, W2)
    acc2 = _sc_aggregate(z2, src2d, dst2d, zerosd)
    z3 = _tc_combine(acc2, z2, dinv, b22, W3)
    acc3 = _sc_aggregate(z3, src2d, dst2d, zerosd)
    return _tc_final(acc3, z3, dinv, b32)[:N]


# speedup vs baseline: 1.2215x; 1.2215x over previous
import functools

import jax
import jax.numpy as jnp
from jax import lax
from jax.experimental import pallas as pl
from jax.experimental.pallas import tpu as pltpu
from jax.experimental.pallas import tpu_sc as plsc

N = 10000
E = 320000
D = 128
NP = 10240

NC = 2
NS = 16
NW = NC * NS
EPW = E // NW
K = 125
ITERS = EPW // K
RPW = NP // NS
HR = NP // D

_mesh = plsc.VectorSubcoreMesh(
    core_axis_name="c", subcore_axis_name="s", num_cores=NC, num_subcores=NS
)


@functools.partial(
    pl.kernel,
    out_type=jax.ShapeDtypeStruct((NC, HR, D), jnp.float32),
    mesh=_mesh,
    compiler_params=pltpu.CompilerParams(needs_layout_passes=False),
    scratch_types=[
        pltpu.VMEM((EPW,), jnp.int32),
        pltpu.VMEM((HR, D), jnp.float32),
        pltpu.VMEM((1, HR), jnp.int32),
        pltpu.VMEM_SHARED((HR, D), jnp.float32),
    ],
)
def _sc_degree(dst_hbm, iota_hbm, zeros_hbm, out_hbm, idx_v, hist_v, iota_v, deg_sh):
    c = lax.axis_index("c")
    s = lax.axis_index("s")
    wid = s * NC + c
    e0 = pl.multiple_of(wid * EPW, 8)
    pltpu.sync_copy(dst_hbm.at[pl.ds(e0, EPW)], idx_v)
    pltpu.sync_copy(zeros_hbm.at[pl.ds(0, HR)], hist_v)
    pltpu.sync_copy(iota_hbm, iota_v)

    @pl.when(s == 0)
    def _():
        pltpu.sync_copy(zeros_hbm.at[pl.ds(0, HR)], deg_sh)

    def body(v, carry):
        idx16 = idx_v[pl.ds(v * 16, 16)]
        row = jnp.right_shift(idx16, 7)
        col = jnp.bitwise_and(idx16, 127)
        cnt, last = plsc.scan_count(idx16)
        plsc.addupdate_scatter(hist_v, [row, col], cnt.astype(jnp.float32), mask=last)
        return carry

    lax.fori_loop(0, EPW // 16, body, 0)
    plsc.subcore_barrier()
    pltpu.sync_copy(hist_v, deg_sh.at[iota_v.at[0]], add=True)
    plsc.subcore_barrier()

    @pl.when(s == 0)
    def _():
        pltpu.sync_copy(deg_sh, out_hbm.at[c])


BC = 8
NB = ITERS // BC
IB = 2
NBUF = 2


@functools.partial(
    pl.kernel,
    out_type=jax.ShapeDtypeStruct((NC, NP, D), jnp.float32),
    mesh=_mesh,
    scratch_types=[
        pltpu.VMEM((IB, BC, K), jnp.int32),
        pltpu.VMEM((IB, BC, K), jnp.int32),
        [pltpu.VMEM((K, D), jnp.float32)] * NBUF,
        [pltpu.SemaphoreType.DMA] * NBUF,
        [pltpu.SemaphoreType.DMA] * NBUF,
        pltpu.VMEM_SHARED((NP, D), jnp.float32),
        pltpu.SemaphoreType.DMA,
    ],
)
def _sc_aggregate(
    z_hbm, src2d_hbm, dst2d_hbm, zerosd_hbm, out_hbm,
    sring, dring, rows, gsems, ssems, acc_sh, isem
):
    c = lax.axis_index("c")
    s = lax.axis_index("s")
    wid = s * NC + c
    chunk0 = pl.multiple_of(wid * ITERS, 8)
    row0 = pl.multiple_of(s * RPW, 8)

    def fire_idx(b, sl):
        blk = pl.multiple_of(chunk0 + b * BC, 8)
        pltpu.async_copy(src2d_hbm.at[pl.ds(blk, BC)], sring.at[sl], isem)
        pltpu.async_copy(dst2d_hbm.at[pl.ds(blk, BC)], dring.at[sl], isem)

    def drain_idx():
        pltpu.make_async_copy(src2d_hbm.at[pl.ds(chunk0, BC)], sring.at[0], isem).wait()
        pltpu.make_async_copy(dst2d_hbm.at[pl.ds(chunk0, BC)], dring.at[0], isem).wait()

    def wait_scatter(q):
        pltpu.make_async_copy(rows[q], acc_sh.at[dring.at[0, 0]], ssems[q]).wait()

    def wait_gather(q):
        pltpu.make_async_copy(z_hbm.at[sring.at[0, 0]], rows[q], gsems[q]).wait()

    pltpu.sync_copy(zerosd_hbm, acc_sh.at[pl.ds(row0, RPW)])
    fire_idx(0, 0)
    fire_idx(1, 1)
    plsc.subcore_barrier()
    drain_idx()
    pltpu.async_copy(z_hbm.at[sring.at[0, 0]], rows[0], gsems[0])

    def block_body(b, carry):
        sl = lax.rem(b, IB)
        nsl = lax.rem(b + 1, IB)

        @pl.when(b + 1 < NB)
        def _():
            drain_idx()

        for r in range(BC):
            q = r % NBUF
            nq = (r + 1) % NBUF

            if r < NBUF - 1:
                @pl.when(b > 0)
                def _():
                    wait_scatter(nq)
            else:
                wait_scatter(nq)
            if r < BC - 1:
                pltpu.async_copy(z_hbm.at[sring.at[sl, r + 1]], rows[nq], gsems[nq])
            else:
                pltpu.async_copy(z_hbm.at[sring.at[nsl, 0]], rows[nq], gsems[nq])
            wait_gather(q)
            pltpu.async_copy(rows[q], acc_sh.at[dring.at[sl, r]], ssems[q], add=True)

        @pl.when(b + 2 < NB)
        def _():
            fire_idx(b + 2, sl)

        return carry

    lax.fori_loop(0, NB, block_body, 0)
    for j in range(ITERS - NBUF + 1, ITERS):
        wait_scatter(j % NBUF)
    wait_gather(ITERS % NBUF)
    plsc.subcore_barrier()
    pltpu.sync_copy(acc_sh.at[pl.ds(row0, RPW)], out_hbm.at[c, pl.ds(row0, RPW)])


BR = 2048

_DOT = dict(precision=lax.Precision.HIGHEST, preferred_element_type=jnp.float32)


def _gelu(x):
    return 0.5 * x * (1.0 + lax.erf(x * 0.7071067811865476))


def _tc_p0_body(latent, wp, bp, w1, y1_out):
    x0 = _gelu(jnp.dot(latent[...], wp[...], **_DOT) + bp[...])
    y1_out[...] = jnp.dot(x0, w1[...], **_DOT)


def _tc_p0(latent, wp, bp, w1):
    return pl.pallas_call(
        _tc_p0_body,
        grid=(NP // BR,),
        in_specs=[
            pl.BlockSpec((BR, D), lambda i: (i, 0)),
            pl.BlockSpec((D, D), lambda i: (0, 0)),
            pl.BlockSpec((1, D), lambda i: (0, 0)),
            pl.BlockSpec((D, D), lambda i: (0, 0)),
        ],
        out_specs=pl.BlockSpec((BR, D), lambda i: (i, 0)),
        out_shape=jax.ShapeDtypeStruct((NP, D), jnp.float32),
    )(latent, wp, bp, w1)


def _tc_p1_body(degcol, y1, z1_out, dinv_out):
    deg = jnp.sum(degcol[...], axis=0) + 1.0
    dinv = lax.rsqrt(deg)
    z1_out[...] = y1[...] * dinv
    dinv_out[...] = jnp.broadcast_to(dinv, (BR, D))


def _tc_p1(degcol, y1):
    return pl.pallas_call(
        _tc_p1_body,
        grid=(NP // BR,),
        in_specs=[
            pl.BlockSpec((NC, BR, 1), lambda i: (0, i, 0)),
            pl.BlockSpec((BR, D), lambda i: (i, 0)),
        ],
        out_specs=[
            pl.BlockSpec((BR, D), lambda i: (i, 0)),
            pl.BlockSpec((BR, D), lambda i: (i, 0)),
        ],
        out_shape=[
            jax.ShapeDtypeStruct((NP, D), jnp.float32),
            jax.ShapeDtypeStruct((NP, D), jnp.float32),
        ],
    )(degcol, y1)


def _tc_combine_body(acc, z, dinv, b, w_next, znext_out):
    x = dinv[...] * (acc[0] + acc[1] + z[...]) + b[...]
    znext_out[...] = jnp.dot(x, w_next[...], **_DOT) * dinv[...]


def _tc_combine(acc, z, dinv, b, w_next):
    return pl.pallas_call(
        _tc_combine_body,
        grid=(NP // BR,),
        in_specs=[
            pl.BlockSpec((NC, BR, D), lambda i: (0, i, 0)),
            pl.BlockSpec((BR, D), lambda i: (i, 0)),
            pl.BlockSpec((BR, D), lambda i: (i, 0)),
            pl.BlockSpec((1, D), lambda i: (0, 0)),
            pl.BlockSpec((D, D), lambda i: (0, 0)),
        ],
        out_specs=pl.BlockSpec((BR, D), lambda i: (i, 0)),
        out_shape=jax.ShapeDtypeStruct((NP, D), jnp.float32),
    )(acc, z, dinv, b, w_next)


def _tc_final_body(acc, z, dinv, b, out):
    out[...] = dinv[...] * (acc[0] + acc[1] + z[...]) + b[...]


def _tc_final(acc, z, dinv, b):
    return pl.pallas_call(
        _tc_final_body,
        grid=(NP // BR,),
        in_specs=[
            pl.BlockSpec((NC, BR, D), lambda i: (0, i, 0)),
            pl.BlockSpec((BR, D), lambda i: (i, 0)),
            pl.BlockSpec((BR, D), lambda i: (i, 0)),
            pl.BlockSpec((1, D), lambda i: (0, 0)),
        ],
        out_specs=pl.BlockSpec((BR, D), lambda i: (i, 0)),
        out_shape=jax.ShapeDtypeStruct((NP, D), jnp.float32),
    )(acc, z, dinv, b)


def kernel(latent, edge_index, Wp, bp, W1, b1, W2, b2, W3, b3):
    src2d = edge_index[0].reshape(E // K, K)
    dst2d = edge_index[1].reshape(E // K, K)
    dstflat = edge_index[1]
    iota_hr = jnp.arange(HR, dtype=jnp.int32).reshape(1, HR)
    zerosd = jnp.zeros((RPW, D), jnp.float32)
    latp = jnp.pad(latent, ((0, NP - N), (0, 0)))
    bp2, b12, b22, b32 = (b.reshape(1, D) for b in (bp, b1, b2, b3))

    degpart = _sc_degree(dstflat, iota_hr, zerosd)
    degcol = degpart.reshape(NC, NP, 1)
    y1 = _tc_p0(latp, Wp, bp2, W1)
    z1, dinv = _tc_p1(degcol, y1)
    acc1 = _sc_aggregate(z1, src2d, dst2d, zerosd)
    z2 = _tc_combine(acc1, z1, dinv, b12, W2)
    acc2 = _sc_aggregate(z2, src2d, dst2d, zerosd)
    z3 = _tc_combine(acc2, z2, dinv, b22, W3)
    acc3 = _sc_aggregate(z3, src2d, dst2d, zerosd)
    return _tc_final(acc3, z3, dinv, b32)[:N]


# speedup vs baseline: 1.2381x; 1.0136x over previous
import functools

import jax
import jax.numpy as jnp
from jax import lax
from jax.experimental import pallas as pl
from jax.experimental.pallas import tpu as pltpu
from jax.experimental.pallas import tpu_sc as plsc

N = 10000
E = 320000
D = 128
NP = 10240

NC = 2
NS = 16
NW = NC * NS
EPW = E // NW
K = 125
ITERS = EPW // K
RPW = NP // NS
HR = NP // D

_mesh = plsc.VectorSubcoreMesh(
    core_axis_name="c", subcore_axis_name="s", num_cores=NC, num_subcores=NS
)


@functools.partial(
    pl.kernel,
    out_type=jax.ShapeDtypeStruct((NC, HR, D), jnp.float32),
    mesh=_mesh,
    compiler_params=pltpu.CompilerParams(needs_layout_passes=False),
    scratch_types=[
        pltpu.VMEM((EPW,), jnp.int32),
        pltpu.VMEM((HR, D), jnp.float32),
        pltpu.VMEM((1, HR), jnp.int32),
        pltpu.VMEM_SHARED((HR, D), jnp.float32),
    ],
)
def _sc_degree(dst_hbm, iota_hbm, zeros_hbm, out_hbm, idx_v, hist_v, iota_v, deg_sh):
    c = lax.axis_index("c")
    s = lax.axis_index("s")
    wid = s * NC + c
    e0 = pl.multiple_of(wid * EPW, 8)
    pltpu.sync_copy(dst_hbm.at[pl.ds(e0, EPW)], idx_v)
    z0 = pl.multiple_of(s * RPW, 8)
    pltpu.sync_copy(zeros_hbm.at[pl.ds(z0, HR)], hist_v)
    pltpu.sync_copy(iota_hbm, iota_v)

    @pl.when(s == 0)
    def _():
        pltpu.sync_copy(zeros_hbm.at[pl.ds(0, HR)], deg_sh)

    def body(v, carry):
        idx16 = idx_v[pl.ds(v * 16, 16)]
        row = jnp.right_shift(idx16, 7)
        col = jnp.bitwise_and(idx16, 127)
        cnt, last = plsc.scan_count(idx16)
        plsc.addupdate_scatter(hist_v, [row, col], cnt.astype(jnp.float32), mask=last)
        return carry

    lax.fori_loop(0, EPW // 16, body, 0)
    plsc.subcore_barrier()
    pltpu.sync_copy(hist_v, deg_sh.at[iota_v.at[0]], add=True)
    plsc.subcore_barrier()

    @pl.when(s == 0)
    def _():
        pltpu.sync_copy(deg_sh, out_hbm.at[c])


BC = 8
NB = ITERS // BC
IB = 2
NBUF = 2


@functools.partial(
    pl.kernel,
    out_type=jax.ShapeDtypeStruct((NC, NP, D), jnp.float32),
    mesh=_mesh,
    scratch_types=[
        pltpu.VMEM((IB, BC, K), jnp.int32),
        pltpu.VMEM((IB, BC, K), jnp.int32),
        [pltpu.VMEM((K, D), jnp.float32)] * NBUF,
        [pltpu.SemaphoreType.DMA] * NBUF,
        [pltpu.SemaphoreType.DMA] * NBUF,
        pltpu.VMEM_SHARED((NP, D), jnp.float32),
        pltpu.SemaphoreType.DMA,
    ],
)
def _sc_aggregate(
    z_hbm, src2d_hbm, dst2d_hbm, zerosd_hbm, out_hbm,
    sring, dring, rows, gsems, ssems, acc_sh, isem
):
    c = lax.axis_index("c")
    s = lax.axis_index("s")
    wid = s * NC + c
    chunk0 = pl.multiple_of(wid * ITERS, 8)
    row0 = pl.multiple_of(s * RPW, 8)

    def fire_idx(b, sl):
        blk = pl.multiple_of(chunk0 + b * BC, 8)
        pltpu.async_copy(src2d_hbm.at[pl.ds(blk, BC)], sring.at[sl], isem)
        pltpu.async_copy(dst2d_hbm.at[pl.ds(blk, BC)], dring.at[sl], isem)

    def drain_idx():
        pltpu.make_async_copy(src2d_hbm.at[pl.ds(chunk0, BC)], sring.at[0], isem).wait()
        pltpu.make_async_copy(dst2d_hbm.at[pl.ds(chunk0, BC)], dring.at[0], isem).wait()

    def wait_scatter(q):
        pltpu.make_async_copy(rows[q], acc_sh.at[dring.at[0, 0]], ssems[q]).wait()

    def wait_gather(q):
        pltpu.make_async_copy(z_hbm.at[sring.at[0, 0]], rows[q], gsems[q]).wait()

    pltpu.sync_copy(zerosd_hbm.at[pl.ds(row0, RPW)], acc_sh.at[pl.ds(row0, RPW)])
    fire_idx(0, 0)
    fire_idx(1, 1)
    plsc.subcore_barrier()
    drain_idx()
    pltpu.async_copy(z_hbm.at[sring.at[0, 0]], rows[0], gsems[0])

    def block_body(b, carry):
        sl = lax.rem(b, IB)
        nsl = lax.rem(b + 1, IB)

        @pl.when(b + 1 < NB)
        def _():
            drain_idx()

        for r in range(BC):
            q = r % NBUF
            nq = (r + 1) % NBUF

            if r < NBUF - 1:
                @pl.when(b > 0)
                def _():
                    wait_scatter(nq)
            else:
                wait_scatter(nq)
            if r < BC - 1:
                pltpu.async_copy(z_hbm.at[sring.at[sl, r + 1]], rows[nq], gsems[nq])
            else:
                pltpu.async_copy(z_hbm.at[sring.at[nsl, 0]], rows[nq], gsems[nq])
            wait_gather(q)
            pltpu.async_copy(rows[q], acc_sh.at[dring.at[sl, r]], ssems[q], add=True)

        @pl.when(b + 2 < NB)
        def _():
            fire_idx(b + 2, sl)

        return carry

    lax.fori_loop(0, NB, block_body, 0)
    for j in range(ITERS - NBUF + 1, ITERS):
        wait_scatter(j % NBUF)
    wait_gather(ITERS % NBUF)
    plsc.subcore_barrier()
    pltpu.sync_copy(acc_sh.at[pl.ds(row0, RPW)], out_hbm.at[c, pl.ds(row0, RPW)])


BR = 2048

_DOT = dict(precision=lax.Precision.HIGHEST, preferred_element_type=jnp.float32)


def _gelu(x):
    return 0.5 * x * (1.0 + lax.erf(x * 0.7071067811865476))


def _tc_proj_body(degcol, latent, wp, bp, w1, z1_out, dinv_out):
    deg = jnp.sum(degcol[...], axis=0) + 1.0
    dinv = lax.rsqrt(deg)
    x0 = _gelu(jnp.dot(latent[...], wp[...], **_DOT) + bp[...])
    z1_out[...] = jnp.dot(x0, w1[...], **_DOT) * dinv
    dinv_out[...] = jnp.broadcast_to(dinv, (BR, D))


def _tc_proj(degcol, latent, wp, bp, w1):
    return pl.pallas_call(
        _tc_proj_body,
        grid=(NP // BR,),
        in_specs=[
            pl.BlockSpec((NC, BR, 1), lambda i: (0, i, 0)),
            pl.BlockSpec((BR, D), lambda i: (i, 0)),
            pl.BlockSpec((D, D), lambda i: (0, 0)),
            pl.BlockSpec((1, D), lambda i: (0, 0)),
            pl.BlockSpec((D, D), lambda i: (0, 0)),
        ],
        out_specs=[
            pl.BlockSpec((BR, D), lambda i: (i, 0)),
            pl.BlockSpec((BR, D), lambda i: (i, 0)),
        ],
        out_shape=[
            jax.ShapeDtypeStruct((NP, D), jnp.float32),
            jax.ShapeDtypeStruct((NP, D), jnp.float32),
        ],
    )(degcol, latent, wp, bp, w1)


def _tc_combine_body(acc, z, dinv, b, w_next, znext_out):
    x = dinv[...] * (acc[0] + acc[1] + z[...]) + b[...]
    znext_out[...] = jnp.dot(x, w_next[...], **_DOT) * dinv[...]


def _tc_combine(acc, z, dinv, b, w_next):
    return pl.pallas_call(
        _tc_combine_body,
        grid=(NP // BR,),
        in_specs=[
            pl.BlockSpec((NC, BR, D), lambda i: (0, i, 0)),
            pl.BlockSpec((BR, D), lambda i: (i, 0)),
            pl.BlockSpec((BR, D), lambda i: (i, 0)),
            pl.BlockSpec((1, D), lambda i: (0, 0)),
            pl.BlockSpec((D, D), lambda i: (0, 0)),
        ],
        out_specs=pl.BlockSpec((BR, D), lambda i: (i, 0)),
        out_shape=jax.ShapeDtypeStruct((NP, D), jnp.float32),
    )(acc, z, dinv, b, w_next)


def _tc_final_body(acc, z, dinv, b, out):
    out[...] = dinv[...] * (acc[0] + acc[1] + z[...]) + b[...]


def _tc_final(acc, z, dinv, b):
    return pl.pallas_call(
        _tc_final_body,
        grid=(NP // BR,),
        in_specs=[
            pl.BlockSpec((NC, BR, D), lambda i: (0, i, 0)),
            pl.BlockSpec((BR, D), lambda i: (i, 0)),
            pl.BlockSpec((BR, D), lambda i: (i, 0)),
            pl.BlockSpec((1, D), lambda i: (0, 0)),
        ],
        out_specs=pl.BlockSpec((BR, D), lambda i: (i, 0)),
        out_shape=jax.ShapeDtypeStruct((NP, D), jnp.float32),
    )(acc, z, dinv, b)


def kernel(latent, edge_index, Wp, bp, W1, b1, W2, b2, W3, b3):
    src2d = edge_index[0].reshape(E // K, K)
    dst2d = edge_index[1].reshape(E // K, K)
    dstflat = edge_index[1]
    iota_hr = jnp.arange(HR, dtype=jnp.int32).reshape(1, HR)
    zerosd = jnp.zeros((NP, D), jnp.float32)
    latp = jnp.pad(latent, ((0, NP - N), (0, 0)))
    bp2, b12, b22, b32 = (b.reshape(1, D) for b in (bp, b1, b2, b3))

    degpart = _sc_degree(dstflat, iota_hr, zerosd)
    degcol = degpart.reshape(NC, NP, 1)
    z1, dinv = _tc_proj(degcol, latp, Wp, bp2, W1)
    acc1 = _sc_aggregate(z1, src2d, dst2d, zerosd)
    z2 = _tc_combine(acc1, z1, dinv, b12, W2)
    acc2 = _sc_aggregate(z2, src2d, dst2d, zerosd)
    z3 = _tc_combine(acc2, z2, dinv, b22, W3)
    acc3 = _sc_aggregate(z3, src2d, dst2d, zerosd)
    return _tc_final(acc3, z3, dinv, b32)[:N]
